# Initial kernel scaffold; baseline (speedup 1.0000x reference)
import functools

import jax
import jax.numpy as jnp
from jax import lax
from jax.experimental import pallas as pl
from jax.experimental.pallas import tpu as pltpu
from jax.experimental.pallas import tpu_sc as plsc

_NC = 2
_NS = 16
_LANES = 16


def _pick_chunk(ept: int) -> int:
    for k in range(128, 7, -1):
        if k % 8 == 0 and ept % k == 0:
            return k
    raise ValueError(f"no valid chunk size for {ept} edges per tile")


def _make_sc_spmm(n: int, d: int, e: int):
    assert e % (_NC * _NS) == 0 and n % _NS == 0 and d % _LANES == 0
    ept = e // (_NC * _NS)
    k = _pick_chunk(ept)
    nchunk = ept // k
    rows_per_sub = n // _NS
    mesh = plsc.VectorSubcoreMesh(core_axis_name="c", subcore_axis_name="s")

    @functools.partial(
        pl.kernel,
        out_type=jax.ShapeDtypeStruct((_NC, n, d), jnp.float32),
        mesh=mesh,
        scratch_types=[
            pltpu.VMEM((k,), jnp.int32),
            pltpu.VMEM((k,), jnp.int32),
            pltpu.VMEM((k,), jnp.float32),
            pltpu.VMEM((k, d), jnp.float32),
            pltpu.VMEM_SHARED((n, d), jnp.float32),
            pltpu.SemaphoreType.DMA,
        ],
    )
    def sc_spmm(x_hbm, src_hbm, dst_hbm, val_hbm, zeros_hbm, out_hbm,
                src_v, dst_v, val_v, rows_v, agg_sh, sem):
        c = lax.axis_index("c")
        s = lax.axis_index("s")
        wid = c * _NS + s
        row0 = s * rows_per_sub
        pltpu.sync_copy(zeros_hbm.at[pl.ds(row0, rows_per_sub)],
                        agg_sh.at[pl.ds(row0, rows_per_sub)])
        plsc.subcore_barrier()

        base = wid * ept

        def chunk_body(i, carry):
            eoff = base + i * k
            pltpu.sync_copy(src_hbm.at[pl.ds(eoff, k)], src_v)
            pltpu.sync_copy(dst_hbm.at[pl.ds(eoff, k)], dst_v)
            pltpu.sync_copy(val_hbm.at[pl.ds(eoff, k)], val_v)
            pltpu.async_copy(x_hbm.at[src_v], rows_v, sem).wait()

            def scale_body(j, carry2):
                v = val_v[j]
                for f in range(d // _LANES):
                    sl = pl.ds(f * _LANES, _LANES)
                    rows_v[j, sl] = rows_v[j, sl] * v
                return carry2

            lax.fori_loop(0, k, scale_body, 0)
            pltpu.sync_copy(rows_v, agg_sh.at[dst_v], add=True)
            return carry

        lax.fori_loop(0, nchunk, chunk_body, 0)
        plsc.subcore_barrier()
        pltpu.sync_copy(agg_sh.at[pl.ds(row0, rows_per_sub)],
                        out_hbm.at[c, pl.ds(row0, rows_per_sub)])

    return sc_spmm


def _dense_body(agg_ref, x_ref, wl_ref, wr_ref, bsum_ref, o_ref):
    a = agg_ref[0] + agg_ref[1]
    h = lax.dot_general(a, wl_ref[...], (((1,), (1,)), ((), ())),
                        preferred_element_type=jnp.float32)
    h = h + lax.dot_general(x_ref[...], wr_ref[...], (((1,), (1,)), ((), ())),
                            preferred_element_type=jnp.float32)
    h = h + bsum_ref[...]
    denom = jnp.maximum(jnp.sum(jnp.abs(h), axis=1, keepdims=True), 1e-12)
    o_ref[...] = h / denom


def _make_dense(n: int, d: int):
    blk = 400
    while n % blk or blk % 8:
        blk //= 2
    grid = n // blk
    return pl.pallas_call(
        _dense_body,
        grid=(grid,),
        in_specs=[
            pl.BlockSpec((_NC, blk, d), lambda i: (0, i, 0)),
            pl.BlockSpec((blk, d), lambda i: (i, 0)),
            pl.BlockSpec((d, d), lambda i: (0, 0)),
            pl.BlockSpec((d, d), lambda i: (0, 0)),
            pl.BlockSpec((1, d), lambda i: (0, 0)),
        ],
        out_specs=pl.BlockSpec((blk, d), lambda i: (i, 0)),
        out_shape=jax.ShapeDtypeStruct((n, d), jnp.float32),
    )


def kernel(x, edge_vals, W_l, b_l, W_r, b_r, edge_index):
    n, d = x.shape
    e = edge_vals.shape[0]
    dst = edge_index[0]
    src = edge_index[1]
    zeros = jnp.zeros((n, d), jnp.float32)
    partials = _make_sc_spmm(n, d, e)(x, src, dst, edge_vals, zeros)
    bsum = (b_l + b_r)[None, :]
    return _make_dense(n, d)(partials, x, W_l, W_r, bsum)


# speedup vs baseline: 4.3906x; 4.3906x over previous
import functools

import jax
import jax.numpy as jnp
from jax import lax
from jax.experimental import pallas as pl
from jax.experimental.pallas import tpu as pltpu
from jax.experimental.pallas import tpu_sc as plsc

_NC = 2
_NS = 16
_LANES = 16


def _pick_chunk(ept: int) -> int:
    for k in range(128, 7, -1):
        if k % 8 == 0 and ept % k == 0:
            return k
    raise ValueError(f"no valid chunk size for {ept} edges per tile")


def _make_sc_spmm(n: int, d: int, e: int):
    assert e % (_NC * _NS) == 0 and n % _NS == 0 and d % _LANES == 0
    ept = e // (_NC * _NS)
    k = _pick_chunk(ept)
    nchunk = ept // k
    row_step = ((n // _NS) // 8) * 8
    row_len = n - (_NS - 1) * row_step
    assert row_len % 8 == 0 and row_len >= row_step
    mesh = plsc.VectorSubcoreMesh(core_axis_name="c", subcore_axis_name="s")

    @functools.partial(
        pl.kernel,
        out_type=jax.ShapeDtypeStruct((_NC, n, d), jnp.float32),
        mesh=mesh,
        scratch_types=[
            pltpu.VMEM((k,), jnp.int32),
            pltpu.VMEM((k,), jnp.int32),
            pltpu.VMEM((k,), jnp.float32),
            pltpu.VMEM((k, d), jnp.float32),
            pltpu.VMEM_SHARED((n, d), jnp.float32),
            pltpu.SemaphoreType.DMA,
        ],
    )
    def sc_spmm(x_hbm, src_hbm, dst_hbm, val_hbm, zeros_hbm, out_hbm,
                src_v, dst_v, val_v, rows_v, agg_sh, sem):
        c = lax.axis_index("c")
        s = lax.axis_index("s")
        wid = c * _NS + s
        row0 = s * row_step
        pltpu.sync_copy(zeros_hbm.at[pl.ds(row0, row_len)],
                        agg_sh.at[pl.ds(row0, row_len)])
        plsc.subcore_barrier()

        base = wid * ept

        def chunk_body(i, carry):
            eoff = base + i * k
            pltpu.sync_copy(src_hbm.at[pl.ds(eoff, k)], src_v)
            pltpu.sync_copy(dst_hbm.at[pl.ds(eoff, k)], dst_v)
            pltpu.sync_copy(val_hbm.at[pl.ds(eoff, k)], val_v)
            pltpu.async_copy(x_hbm.at[src_v], rows_v, sem).wait()

            def scale_body(g, carry2):
                vv = val_v[pl.ds(g * _LANES, _LANES)]
                for t in range(_LANES):
                    v = vv[t]
                    r = g * _LANES + t
                    for f in range(d // _LANES):
                        sl = pl.ds(f * _LANES, _LANES)
                        rows_v[r, sl] = rows_v[r, sl] * v
                return carry2

            lax.fori_loop(0, k // _LANES, scale_body, 0)
            pltpu.sync_copy(rows_v, agg_sh.at[dst_v], add=True)
            return carry

        lax.fori_loop(0, nchunk, chunk_body, 0)
        plsc.subcore_barrier()
        pltpu.sync_copy(agg_sh.at[pl.ds(row0, row_len)],
                        out_hbm.at[c, pl.ds(row0, row_len)])

    return sc_spmm


def _dense_body(agg_ref, x_ref, wl_ref, wr_ref, bsum_ref, o_ref):
    a = agg_ref[0] + agg_ref[1]
    h = lax.dot_general(a, wl_ref[...], (((1,), (1,)), ((), ())),
                        preferred_element_type=jnp.float32)
    h = h + lax.dot_general(x_ref[...], wr_ref[...], (((1,), (1,)), ((), ())),
                            preferred_element_type=jnp.float32)
    h = h + bsum_ref[...]
    denom = jnp.maximum(jnp.sum(jnp.abs(h), axis=1, keepdims=True), 1e-12)
    o_ref[...] = h / denom


def _make_dense(n: int, d: int):
    blk = 400
    while n % blk or blk % 8:
        blk //= 2
    grid = n // blk
    return pl.pallas_call(
        _dense_body,
        grid=(grid,),
        in_specs=[
            pl.BlockSpec((_NC, blk, d), lambda i: (0, i, 0)),
            pl.BlockSpec((blk, d), lambda i: (i, 0)),
            pl.BlockSpec((d, d), lambda i: (0, 0)),
            pl.BlockSpec((d, d), lambda i: (0, 0)),
            pl.BlockSpec((1, d), lambda i: (0, 0)),
        ],
        out_specs=pl.BlockSpec((blk, d), lambda i: (i, 0)),
        out_shape=jax.ShapeDtypeStruct((n, d), jnp.float32),
    )


def kernel(x, edge_vals, W_l, b_l, W_r, b_r, edge_index):
    n, d = x.shape
    e = edge_vals.shape[0]
    dst = edge_index[0]
    src = edge_index[1]
    zeros = jnp.zeros((n, d), jnp.float32)
    partials = _make_sc_spmm(n, d, e)(x, src, dst, edge_vals, zeros)
    bsum = (b_l + b_r)[None, :]
    return _make_dense(n, d)(partials, x, W_l, W_r, bsum)
